# Initial kernel scaffold; baseline (speedup 1.0000x reference)
import jax
import jax.numpy as jnp
from jax.experimental import pallas as pl


def kernel(x, edge_index0, edge_index1, W1, al1, ar1, b1, W2, al2, ar2, b2, W3, al3, ar3, b3, W4, al4, ar4, b4):
    raise NotImplementedError("write your pallas kernel here")


# speedup vs baseline: 2.1271x; 2.1271x over previous
import functools

import jax
import jax.numpy as jnp
from jax import lax
from jax.experimental import pallas as pl
from jax.experimental.pallas import tpu as pltpu

N = 10000
E = 640000
ROW_BLK = 1000


def _dense_body(x_ref, w_ref, al_ref, ar_ref, o_ref, *, cpad):
    h = jnp.dot(x_ref[...], w_ref[...], preferred_element_type=jnp.float32)
    el = jnp.sum(h * al_ref[...], axis=1, keepdims=True)
    er = jnp.sum(h * ar_ref[...], axis=1, keepdims=True)
    pad = jnp.zeros((h.shape[0], cpad - h.shape[1] - 2), jnp.float32)
    o_ref[...] = jnp.concatenate([h, el, er, pad], axis=-1)


def _dense_stage(x, W, al, ar, cpad):
    din, dout = W.shape
    grid = (N // ROW_BLK,)
    return pl.pallas_call(
        functools.partial(_dense_body, cpad=cpad),
        grid=grid,
        in_specs=[
            pl.BlockSpec((ROW_BLK, din), lambda i: (i, 0)),
            pl.BlockSpec((din, dout), lambda i: (0, 0)),
            pl.BlockSpec((1, dout), lambda i: (0, 0)),
            pl.BlockSpec((1, dout), lambda i: (0, 0)),
        ],
        out_specs=pl.BlockSpec((ROW_BLK, cpad), lambda i: (i, 0)),
        out_shape=jax.ShapeDtypeStruct((N, cpad), jnp.float32),
    )(x, W, al.reshape(1, dout), ar.reshape(1, dout))


def _finish_body(acc_ref, den_ref, b_ref, o_ref):
    o_ref[...] = acc_ref[...] / (den_ref[...] + 1e-9) + b_ref[...]


def _finish_stage(acc, denom, b, dout):
    grid = (N // ROW_BLK,)
    return pl.pallas_call(
        _finish_body,
        grid=grid,
        in_specs=[
            pl.BlockSpec((ROW_BLK, dout), lambda i: (i, 0)),
            pl.BlockSpec((ROW_BLK, 1), lambda i: (i, 0)),
            pl.BlockSpec((1, dout), lambda i: (0, 0)),
        ],
        out_specs=pl.BlockSpec((ROW_BLK, dout), lambda i: (i, 0)),
        out_shape=jax.ShapeDtypeStruct((N, dout), jnp.float32),
    )(acc[:, :dout], denom.reshape(N, 1), b.reshape(1, dout))


def _edge_softmax_agg_jnp(table, src, dst, dout, dcol):
    h = table[:, :dout]
    el = table[:, dcol]
    er = table[:, dcol + 1]
    e = jax.nn.leaky_relu(el[src] + er[dst], 0.2)
    gmax = jnp.max(e)
    ee = jnp.exp(e - gmax)
    denom = jax.ops.segment_sum(ee, dst, num_segments=N)
    acc = jax.ops.segment_sum(h[src] * ee[:, None], dst, num_segments=N)
    return acc, denom


def kernel(x, edge_index0, edge_index1, W1, al1, ar1, b1, W2, al2, ar2, b2,
           W3, al3, ar3, b3, W4, al4, ar4, b4):
    src0 = edge_index0[0].astype(jnp.int32)
    dst0 = edge_index0[1].astype(jnp.int32)
    src1 = edge_index1[0].astype(jnp.int32)
    dst1 = edge_index1[1].astype(jnp.int32)

    t0 = _dense_stage(x, W1, al1, ar1, 64)
    t1 = _dense_stage(x, W3, al3, ar3, 64)

    acc1, den1 = _edge_softmax_agg_jnp(t0, src0, dst0, 60, 60)
    acc3, den3 = _edge_softmax_agg_jnp(t1, src0, dst0, 60, 60)

    out1 = _finish_stage(acc1, den1, b1, 60)
    out3 = _finish_stage(acc3, den3, b3, 60)

    t2 = _dense_stage(out1, W2, al2, ar2, 128)
    t4 = _dense_stage(out3, W4, al4, ar4, 128)

    acc2, den2 = _edge_softmax_agg_jnp(t2, src1, dst1, 120, 120)
    acc4, den4 = _edge_softmax_agg_jnp(t4, src1, dst1, 120, 120)

    h_a = _finish_stage(acc2, den2, b2, 120)
    h_f = _finish_stage(acc4, den4, b4, 120)
    return (h_a, h_f)


# speedup vs baseline: 46.1599x; 21.7006x over previous
import functools

import jax
import jax.numpy as jnp
from jax import lax
from jax.experimental import pallas as pl
from jax.experimental.pallas import tpu as pltpu
from jax.experimental.pallas import tpu_sc as plsc

N = 10000
E = 640000
NPAD = 10240
EPAD = 655360
ROW_BLK = 1024
_PADROWS = 240


def _dense_body(x_ref, w_ref, al_ref, ar_ref, o_ref, m_ref, *, cpad):
    i = pl.program_id(0)
    h = jnp.dot(x_ref[...], w_ref[...], preferred_element_type=jnp.float32)
    el = jnp.sum(h * al_ref[...], axis=1, keepdims=True)
    er = jnp.sum(h * ar_ref[...], axis=1, keepdims=True)
    pad = jnp.zeros((h.shape[0], cpad - h.shape[1] - 2), jnp.float32)
    o_ref[...] = jnp.concatenate([h, el, er, pad], axis=-1)
    cols = lax.broadcasted_iota(jnp.int32, (1, 128), 1)
    cur = jnp.where(cols == 0, jnp.max(el), jnp.where(cols == 1, jnp.max(er), -jnp.inf))

    @pl.when(i == 0)
    def _():
        m_ref[...] = cur

    @pl.when(i > 0)
    def _():
        m_ref[...] = jnp.maximum(m_ref[...], cur)


def _dense_stage(x, W, al, ar, cpad):
    din, dout = W.shape
    return pl.pallas_call(
        functools.partial(_dense_body, cpad=cpad),
        grid=(NPAD // ROW_BLK,),
        in_specs=[
            pl.BlockSpec((ROW_BLK, din), lambda i: (i, 0)),
            pl.BlockSpec((din, dout), lambda i: (0, 0)),
            pl.BlockSpec((1, dout), lambda i: (0, 0)),
            pl.BlockSpec((1, dout), lambda i: (0, 0)),
        ],
        out_specs=[
            pl.BlockSpec((ROW_BLK, cpad), lambda i: (i, 0)),
            pl.BlockSpec((1, 128), lambda i: (0, 0)),
        ],
        out_shape=[
            jax.ShapeDtypeStruct((NPAD, cpad), jnp.float32),
            jax.ShapeDtypeStruct((1, 128), jnp.float32),
        ],
    )(x, W, al.reshape(1, dout), ar.reshape(1, dout))


def _finish_body(acc_ref, den_ref, b_ref, o_ref):
    o_ref[...] = acc_ref[...] / (den_ref[...] + 1e-9) + b_ref[...]


def _finish_stage(acc, denom, b, dout):
    return pl.pallas_call(
        _finish_body,
        grid=(NPAD // ROW_BLK,),
        in_specs=[
            pl.BlockSpec((ROW_BLK, dout), lambda i: (i, 0)),
            pl.BlockSpec((ROW_BLK, 1), lambda i: (i, 0)),
            pl.BlockSpec((1, dout), lambda i: (0, 0)),
        ],
        out_specs=pl.BlockSpec((ROW_BLK, dout), lambda i: (i, 0)),
        out_shape=jax.ShapeDtypeStruct((NPAD, dout), jnp.float32),
    )(acc[:, :dout], denom.reshape(NPAD, 1), b.reshape(1, dout))


def _make_edge_sc(C):
    G = 2 if C <= 64 else 1
    CH = 128 * G
    RPT = (EPAD // 128) // 16
    NCH = RPT // G
    CN = C // 16
    RT = NPAD // 16
    f32, i32 = jnp.float32, jnp.int32
    mesh = plsc.VectorSubcoreMesh(core_axis_name="c", subcore_axis_name="s")

    @functools.partial(
        pl.kernel,
        out_type=(
            jax.ShapeDtypeStruct((2, NPAD, C), f32),
            jax.ShapeDtypeStruct((2, NPAD), f32),
        ),
        mesh=mesh,
        compiler_params=pltpu.CompilerParams(needs_layout_passes=False,
                                             use_tc_tiling_on_sc=False),
        scratch_types=[
            pltpu.VMEM((NPAD,), f32),
            pltpu.VMEM((NPAD,), f32),
            pltpu.VMEM((G, 128), i32),
            pltpu.VMEM((G, 128), i32),
            pltpu.VMEM((G, 128), f32),
            pltpu.VMEM((CH, C), f32),
            pltpu.VMEM((16,), f32),
            pltpu.VMEM((RT,), f32),
            pltpu.VMEM_SHARED((NPAD, C), f32),
            pltpu.VMEM_SHARED((NPAD,), f32),
            pltpu.SemaphoreType.DMA,
        ],
    )
    def k(table, elers, bvec, src2, dst3, acc_o, den_o,
          el_v, er_v, src_v, dst_v, ee_v, rows_v, bv_v, zv, acc_s, den_s, sem):
        cid = lax.axis_index("c")
        sid = lax.axis_index("s")

        def zero_b(i, c):
            zv[pl.ds(i * 16, 16)] = jnp.zeros((16,), f32)
            return c
        lax.fori_loop(0, RT // 16, zero_b, 0)

        def zero_rows(i, c):
            for cc in range(CN):
                rows_v[i, pl.ds(cc * 16, 16)] = jnp.zeros((16,), f32)
            return c
        lax.fori_loop(0, CH, zero_rows, 0)

        nfull = RT // CH
        for j in range(nfull):
            pltpu.sync_copy(rows_v, acc_s.at[pl.ds(sid * RT + j * CH, CH)])
        rem = RT - nfull * CH
        if rem:
            pltpu.sync_copy(rows_v.at[pl.ds(0, rem)],
                            acc_s.at[pl.ds(sid * RT + nfull * CH, rem)])
        pltpu.sync_copy(zv, den_s.at[pl.ds(sid * RT, RT)])

        pltpu.sync_copy(elers.at[cid].at[0], el_v)
        pltpu.sync_copy(elers.at[cid].at[1], er_v)
        pltpu.sync_copy(bvec.at[cid], bv_v)
        plsc.subcore_barrier()

        bv = bv_v[...]
        tbl = table.at[cid]

        def chunk(g, carry):
            row0 = sid * RPT + g * G
            pltpu.sync_copy(src2.at[pl.ds(row0, G)], src_v)
            pltpu.sync_copy(dst3.at[pl.ds(row0, G)], dst_v)
            for r in range(G):
                pltpu.async_copy(tbl.at[src_v.at[r]],
                                 rows_v.at[pl.ds(r * 128, 128)], sem).wait()

                def ee_b(kk, c):
                    sv = src_v[r, pl.ds(kk * 16, 16)]
                    dv = dst_v[r, pl.ds(kk * 16, 16)]
                    elg = plsc.load_gather(el_v, [sv])
                    erg = plsc.load_gather(er_v, [dv])
                    z = elg + erg
                    z = jnp.maximum(z, z * 0.2)
                    ee_v[r, pl.ds(kk * 16, 16)] = jnp.exp(z - bv)
                    return c
                lax.fori_loop(0, 8, ee_b, 0)

                def scale_b(kk, c):
                    eev = ee_v[r, pl.ds(kk * 16, 16)]
                    for j in range(16):
                        s = eev[j]
                        row = r * 128 + kk * 16 + j
                        for cc in range(CN):
                            rows_v[row, pl.ds(cc * 16, 16)] = (
                                rows_v[row, pl.ds(cc * 16, 16)] * s)
                    return c
                lax.fori_loop(0, 8, scale_b, 0)

                pltpu.sync_copy(rows_v.at[pl.ds(r * 128, 128)],
                                acc_s.at[dst_v.at[r]], add=True)
                pltpu.sync_copy(ee_v.at[r], den_s.at[dst_v.at[r]], add=True)
            return carry
        lax.fori_loop(0, NCH, chunk, 0)

        plsc.subcore_barrier()
        pltpu.sync_copy(acc_s.at[pl.ds(sid * RT, RT)],
                        acc_o.at[cid].at[pl.ds(sid * RT, RT)])
        pltpu.sync_copy(den_s.at[pl.ds(sid * RT, RT)],
                        den_o.at[cid].at[pl.ds(sid * RT, RT)])

    return k


_edge_sc_64 = _make_edge_sc(64)
_edge_sc_128 = _make_edge_sc(128)


def _prep_edges(src, dst):
    pad = EPAD - E
    src_p = jnp.concatenate([src, (jnp.arange(pad, dtype=jnp.int32) * 97) % N])
    dst_p = jnp.concatenate([dst, N + jnp.arange(pad, dtype=jnp.int32) % _PADROWS])
    return src_p.reshape(EPAD // 128, 128), dst_p.reshape(EPAD // 128, 128)


def _bound_vec(m):
    b = m[0, 0] + m[0, 1]
    b = jnp.maximum(b, 0.2 * b)
    return jnp.full((16,), b, jnp.float32)


def kernel(x, edge_index0, edge_index1, W1, al1, ar1, b1, W2, al2, ar2, b2,
           W3, al3, ar3, b3, W4, al4, ar4, b4):
    src0, dst0 = _prep_edges(edge_index0[0].astype(jnp.int32),
                             edge_index0[1].astype(jnp.int32))
    src1, dst1 = _prep_edges(edge_index1[0].astype(jnp.int32),
                             edge_index1[1].astype(jnp.int32))
    xp = jnp.pad(x, ((0, NPAD - N), (0, 0)))

    t0, m1 = _dense_stage(xp, W1, al1, ar1, 64)
    t1, m3 = _dense_stage(xp, W3, al3, ar3, 64)
    elers_a = jnp.stack([jnp.stack([t0[:, 60], t0[:, 61]]),
                         jnp.stack([t1[:, 60], t1[:, 61]])])
    bvec_a = jnp.stack([_bound_vec(m1), _bound_vec(m3)])
    acc_a, den_a = _edge_sc_64(jnp.stack([t0, t1]), elers_a, bvec_a, src0, dst0)

    out1 = _finish_stage(acc_a[0], den_a[0], b1, 60)
    out3 = _finish_stage(acc_a[1], den_a[1], b3, 60)

    t2, m2 = _dense_stage(out1, W2, al2, ar2, 128)
    t4, m4 = _dense_stage(out3, W4, al4, ar4, 128)
    elers_b = jnp.stack([jnp.stack([t2[:, 120], t2[:, 121]]),
                         jnp.stack([t4[:, 120], t4[:, 121]])])
    bvec_b = jnp.stack([_bound_vec(m2), _bound_vec(m4)])
    acc_b, den_b = _edge_sc_128(jnp.stack([t2, t4]), elers_b, bvec_b, src1, dst1)

    h_a = _finish_stage(acc_b[0], den_b[0], b2, 120)
    h_f = _finish_stage(acc_b[1], den_b[1], b4, 120)
    return (h_a[:N], h_f[:N])


# speedup vs baseline: 61.3472x; 1.3290x over previous
import functools

import jax
import jax.numpy as jnp
from jax import lax
from jax.experimental import pallas as pl
from jax.experimental.pallas import tpu as pltpu
from jax.experimental.pallas import tpu_sc as plsc

N = 10000
E = 640000
NPAD = 10240
EPAD = 655360
ROW_BLK = 1024
_PADROWS = 240


def _dense_body(x_ref, w_ref, al_ref, ar_ref, o_ref, m_ref, *, cpad):
    i = pl.program_id(0)
    h = jnp.dot(x_ref[...], w_ref[...], preferred_element_type=jnp.float32)
    el = jnp.sum(h * al_ref[...], axis=1, keepdims=True)
    er = jnp.sum(h * ar_ref[...], axis=1, keepdims=True)
    pad = jnp.zeros((h.shape[0], cpad - h.shape[1] - 2), jnp.float32)
    o_ref[...] = jnp.concatenate([h, el, er, pad], axis=-1)
    cols = lax.broadcasted_iota(jnp.int32, (1, 128), 1)
    cur = jnp.where(cols == 0, jnp.max(el), jnp.where(cols == 1, jnp.max(er), -jnp.inf))

    @pl.when(i == 0)
    def _():
        m_ref[...] = cur

    @pl.when(i > 0)
    def _():
        m_ref[...] = jnp.maximum(m_ref[...], cur)


def _dense_stage(x, W, al, ar, cpad):
    din, dout = W.shape
    return pl.pallas_call(
        functools.partial(_dense_body, cpad=cpad),
        grid=(NPAD // ROW_BLK,),
        in_specs=[
            pl.BlockSpec((ROW_BLK, din), lambda i: (i, 0)),
            pl.BlockSpec((din, dout), lambda i: (0, 0)),
            pl.BlockSpec((1, dout), lambda i: (0, 0)),
            pl.BlockSpec((1, dout), lambda i: (0, 0)),
        ],
        out_specs=[
            pl.BlockSpec((ROW_BLK, cpad), lambda i: (i, 0)),
            pl.BlockSpec((1, 128), lambda i: (0, 0)),
        ],
        out_shape=[
            jax.ShapeDtypeStruct((NPAD, cpad), jnp.float32),
            jax.ShapeDtypeStruct((1, 128), jnp.float32),
        ],
    )(x, W, al.reshape(1, dout), ar.reshape(1, dout))


def _finish_body(acc_ref, den_ref, b_ref, o_ref):
    o_ref[...] = acc_ref[...] / (den_ref[...] + 1e-9) + b_ref[...]


def _finish_stage(acc, denom, b, dout):
    return pl.pallas_call(
        _finish_body,
        grid=(NPAD // ROW_BLK,),
        in_specs=[
            pl.BlockSpec((ROW_BLK, dout), lambda i: (i, 0)),
            pl.BlockSpec((ROW_BLK, 1), lambda i: (i, 0)),
            pl.BlockSpec((1, dout), lambda i: (0, 0)),
        ],
        out_specs=pl.BlockSpec((ROW_BLK, dout), lambda i: (i, 0)),
        out_shape=jax.ShapeDtypeStruct((NPAD, dout), jnp.float32),
    )(acc[:, :dout], denom.reshape(NPAD, 1), b.reshape(1, dout))


def _make_edge_sc(C):
    GG = 128 if C <= 64 else 64
    NG = (EPAD // GG) // 16
    KN = GG // 16
    CN = C // 16
    RT = NPAD // 16
    f32, i32 = jnp.float32, jnp.int32
    mesh = plsc.VectorSubcoreMesh(core_axis_name="c", subcore_axis_name="s")

    @functools.partial(
        pl.kernel,
        out_type=(
            jax.ShapeDtypeStruct((2, NPAD, C), f32),
            jax.ShapeDtypeStruct((2, NPAD), f32),
        ),
        mesh=mesh,
        compiler_params=pltpu.CompilerParams(needs_layout_passes=False,
                                             use_tc_tiling_on_sc=False),
        scratch_types=[
            pltpu.VMEM((NPAD,), f32),
            pltpu.VMEM((NPAD,), f32),
            pltpu.VMEM((4, 2, GG), i32),
            pltpu.VMEM((2, GG), f32),
            pltpu.VMEM((2, GG, C), f32),
            pltpu.VMEM((16,), f32),
            pltpu.VMEM((RT,), f32),
            pltpu.VMEM_SHARED((NPAD, C), f32),
            pltpu.VMEM_SHARED((NPAD,), f32),
            pltpu.SemaphoreType.DMA,
            pltpu.SemaphoreType.DMA,
            pltpu.SemaphoreType.DMA,
            pltpu.SemaphoreType.DMA,
            pltpu.SemaphoreType.DMA,
        ],
    )
    def k(table, elers, bvec, sd_in, acc_o, den_o,
          el_v, er_v, sd_v, ee_v, rows_v, bv_v, zv, acc_s, den_s,
          isem, gsem0, gsem1, ssem0, ssem1):
        cid = lax.axis_index("c")
        sid = lax.axis_index("s")
        gsem = (gsem0, gsem1)
        ssem = (ssem0, ssem1)

        def zero_b(i, c):
            zv[pl.ds(i * 16, 16)] = jnp.zeros((16,), f32)
            return c
        lax.fori_loop(0, RT // 16, zero_b, 0)

        def zero_rows(i, c):
            for cc in range(CN):
                rows_v[0, i, pl.ds(cc * 16, 16)] = jnp.zeros((16,), f32)
            return c
        lax.fori_loop(0, GG, zero_rows, 0)

        for j in range(RT // GG):
            pltpu.sync_copy(rows_v.at[0], acc_s.at[pl.ds(sid * RT + j * GG, GG)])
        pltpu.sync_copy(zv, den_s.at[pl.ds(sid * RT, RT)])

        pltpu.sync_copy(elers.at[cid].at[0], el_v)
        pltpu.sync_copy(elers.at[cid].at[1], er_v)
        pltpu.sync_copy(bvec.at[cid], bv_v)
        plsc.subcore_barrier()

        bv = bv_v[...]
        tbl = table.at[cid]
        base = sid * NG

        def ee_compute(p, b):
            def ee_b(kk, c):
                sv = sd_v[p, 0, pl.ds(kk * 16, 16)]
                dv = sd_v[p, 1, pl.ds(kk * 16, 16)]
                elg = plsc.load_gather(el_v, [sv])
                erg = plsc.load_gather(er_v, [dv])
                z = elg + erg
                z = jnp.maximum(z, z * 0.2)
                ee_v[b, pl.ds(kk * 16, 16)] = jnp.exp(z - bv)
                return c
            lax.fori_loop(0, KN, ee_b, 0)

        def scale(b):
            def scale_b(kk, c):
                eev = ee_v[b, pl.ds(kk * 16, 16)]
                for j in range(16):
                    s = eev[j]
                    row = kk * 16 + j
                    for cc in range(CN):
                        rows_v[b, row, pl.ds(cc * 16, 16)] = (
                            rows_v[b, row, pl.ds(cc * 16, 16)] * s)
                return c
            lax.fori_loop(0, KN, scale_b, 0)

        def scatter_descs(p, b):
            rowd = pltpu.make_async_copy(rows_v.at[b],
                                         acc_s.at[sd_v.at[p].at[1]], ssem[b])
            dend = pltpu.make_async_copy(ee_v.at[b],
                                         den_s.at[sd_v.at[p].at[1]], ssem[b])
            return rowd, dend

        def process(g, gp1_valid, drain_prev, b):
            nb = 1 - b
            for p in (b, b + 2):
                pq = (p + 1) % 4

                @pl.when(g % 4 == p)
                def _():
                    @pl.when(gp1_valid)
                    def _():
                        pltpu.async_copy(sd_in.at[base + g + 1], sd_v.at[pq],
                                         isem)
                    ee_compute(p, b)
                    pltpu.make_async_copy(tbl.at[sd_v.at[p].at[0]],
                                          rows_v.at[b], gsem[b]).wait()
                    scale(b)

                    @pl.when(drain_prev)
                    def _():
                        rd, dd = scatter_descs((p + 3) % 4, nb)
                        rd.wait()
                        dd.wait()

                    @pl.when(gp1_valid)
                    def _():
                        pltpu.make_async_copy(sd_in.at[base + g + 1],
                                              sd_v.at[pq], isem).wait()
                        pltpu.async_copy(tbl.at[sd_v.at[pq].at[0]],
                                         rows_v.at[nb], gsem[nb])

                    pltpu.async_copy(rows_v.at[b], acc_s.at[sd_v.at[p].at[1]],
                                     ssem[b], add=True)
                    pltpu.async_copy(ee_v.at[b], den_s.at[sd_v.at[p].at[1]],
                                     ssem[b], add=True)

        pltpu.sync_copy(sd_in.at[base], sd_v.at[0])
        pltpu.async_copy(tbl.at[sd_v.at[0].at[0]], rows_v.at[0], gsem[0])

        def pair(i, carry):
            g0 = i * 2
            process(g0, jnp.bool_(True), i > 0, 0)
            process(g0 + 1, i < NG // 2 - 1, jnp.bool_(True), 1)
            return carry
        lax.fori_loop(0, NG // 2, pair, 0)

        rd, dd = scatter_descs((NG - 1) % 4, 1)
        rd.wait()
        dd.wait()

        plsc.subcore_barrier()
        pltpu.sync_copy(acc_s.at[pl.ds(sid * RT, RT)],
                        acc_o.at[cid].at[pl.ds(sid * RT, RT)])
        pltpu.sync_copy(den_s.at[pl.ds(sid * RT, RT)],
                        den_o.at[cid].at[pl.ds(sid * RT, RT)])

    return k


_edge_sc_64 = _make_edge_sc(64)
_edge_sc_128 = _make_edge_sc(128)


def _prep_edges(src, dst, gg):
    pad = EPAD - E
    src_p = jnp.concatenate([src, (jnp.arange(pad, dtype=jnp.int32) * 97) % N])
    dst_p = jnp.concatenate([dst, N + jnp.arange(pad, dtype=jnp.int32) % _PADROWS])
    return jnp.stack([src_p.reshape(EPAD // gg, gg),
                      dst_p.reshape(EPAD // gg, gg)], axis=1)


def _bound_vec(m):
    b = m[0, 0] + m[0, 1]
    b = jnp.maximum(b, 0.2 * b)
    return jnp.full((16,), b, jnp.float32)


def kernel(x, edge_index0, edge_index1, W1, al1, ar1, b1, W2, al2, ar2, b2,
           W3, al3, ar3, b3, W4, al4, ar4, b4):
    sd0 = _prep_edges(edge_index0[0].astype(jnp.int32),
                      edge_index0[1].astype(jnp.int32), 128)
    sd1 = _prep_edges(edge_index1[0].astype(jnp.int32),
                      edge_index1[1].astype(jnp.int32), 64)
    xp = jnp.pad(x, ((0, NPAD - N), (0, 0)))

    t0, m1 = _dense_stage(xp, W1, al1, ar1, 64)
    t1, m3 = _dense_stage(xp, W3, al3, ar3, 64)
    elers_a = jnp.stack([jnp.stack([t0[:, 60], t0[:, 61]]),
                         jnp.stack([t1[:, 60], t1[:, 61]])])
    bvec_a = jnp.stack([_bound_vec(m1), _bound_vec(m3)])
    acc_a, den_a = _edge_sc_64(jnp.stack([t0, t1]), elers_a, bvec_a, sd0)

    out1 = _finish_stage(acc_a[0], den_a[0], b1, 60)
    out3 = _finish_stage(acc_a[1], den_a[1], b3, 60)

    t2, m2 = _dense_stage(out1, W2, al2, ar2, 128)
    t4, m4 = _dense_stage(out3, W4, al4, ar4, 128)
    elers_b = jnp.stack([jnp.stack([t2[:, 120], t2[:, 121]]),
                         jnp.stack([t4[:, 120], t4[:, 121]])])
    bvec_b = jnp.stack([_bound_vec(m2), _bound_vec(m4)])
    acc_b, den_b = _edge_sc_128(jnp.stack([t2, t4]), elers_b, bvec_b, sd1)

    h_a = _finish_stage(acc_b[0], den_b[0], b2, 120)
    h_f = _finish_stage(acc_b[1], den_b[1], b4, 120)
    return (h_a[:N], h_f[:N])


# speedup vs baseline: 62.3316x; 1.0160x over previous
import functools

import jax
import jax.numpy as jnp
from jax import lax
from jax.experimental import pallas as pl
from jax.experimental.pallas import tpu as pltpu
from jax.experimental.pallas import tpu_sc as plsc

N = 10000
E = 640000
NPAD = 10240
EPAD = 655360
ROW_BLK = 1024
_PADROWS = 240


def _dense_body(x_ref, w_ref, al_ref, ar_ref, o_ref, m_ref, *, cpad):
    i = pl.program_id(0)
    h = jnp.dot(x_ref[...], w_ref[...], preferred_element_type=jnp.float32)
    el = jnp.sum(h * al_ref[...], axis=1, keepdims=True)
    er = jnp.sum(h * ar_ref[...], axis=1, keepdims=True)
    pad = jnp.zeros((h.shape[0], cpad - h.shape[1] - 2), jnp.float32)
    o_ref[...] = jnp.concatenate([h, el, er, pad], axis=-1)
    cols = lax.broadcasted_iota(jnp.int32, (1, 128), 1)
    cur = jnp.where(cols == 0, jnp.max(el), jnp.where(cols == 1, jnp.max(er), -jnp.inf))

    @pl.when(i == 0)
    def _():
        m_ref[...] = cur

    @pl.when(i > 0)
    def _():
        m_ref[...] = jnp.maximum(m_ref[...], cur)


def _dense_stage(x, W, al, ar, cpad):
    din, dout = W.shape
    return pl.pallas_call(
        functools.partial(_dense_body, cpad=cpad),
        grid=(NPAD // ROW_BLK,),
        in_specs=[
            pl.BlockSpec((ROW_BLK, din), lambda i: (i, 0)),
            pl.BlockSpec((din, dout), lambda i: (0, 0)),
            pl.BlockSpec((1, dout), lambda i: (0, 0)),
            pl.BlockSpec((1, dout), lambda i: (0, 0)),
        ],
        out_specs=[
            pl.BlockSpec((ROW_BLK, cpad), lambda i: (i, 0)),
            pl.BlockSpec((1, 128), lambda i: (0, 0)),
        ],
        out_shape=[
            jax.ShapeDtypeStruct((NPAD, cpad), jnp.float32),
            jax.ShapeDtypeStruct((1, 128), jnp.float32),
        ],
    )(x, W, al.reshape(1, dout), ar.reshape(1, dout))


def _finish_body(acc_ref, den_ref, b_ref, o_ref):
    o_ref[...] = acc_ref[...] / (den_ref[...] + 1e-9) + b_ref[...]


def _finish_stage(acc, denom, b, dout):
    return pl.pallas_call(
        _finish_body,
        grid=(NPAD // ROW_BLK,),
        in_specs=[
            pl.BlockSpec((ROW_BLK, dout), lambda i: (i, 0)),
            pl.BlockSpec((ROW_BLK, 1), lambda i: (i, 0)),
            pl.BlockSpec((1, dout), lambda i: (0, 0)),
        ],
        out_specs=pl.BlockSpec((ROW_BLK, dout), lambda i: (i, 0)),
        out_shape=jax.ShapeDtypeStruct((NPAD, dout), jnp.float32),
    )(acc[:, :dout], denom.reshape(NPAD, 1), b.reshape(1, dout))


def _make_edge_sc(C):
    GG = 128 if C <= 64 else 64
    NG = (EPAD // GG) // 16
    KN = GG // 16
    CN = C // 16
    RT = NPAD // 16
    f32, i32 = jnp.float32, jnp.int32
    mesh = plsc.VectorSubcoreMesh(core_axis_name="c", subcore_axis_name="s")

    @functools.partial(
        pl.kernel,
        out_type=(
            jax.ShapeDtypeStruct((2, NPAD, C), f32),
            jax.ShapeDtypeStruct((2, NPAD), f32),
        ),
        mesh=mesh,
        compiler_params=pltpu.CompilerParams(needs_layout_passes=False,
                                             use_tc_tiling_on_sc=False),
        scratch_types=[
            pltpu.VMEM((NPAD,), f32),
            pltpu.VMEM((NPAD,), f32),
            pltpu.VMEM((4, 2, GG), i32),
            pltpu.VMEM((2, GG), f32),
            pltpu.VMEM((2, GG, C), f32),
            pltpu.VMEM((16,), f32),
            pltpu.VMEM((RT,), f32),
            pltpu.VMEM_SHARED((NPAD, C), f32),
            pltpu.VMEM_SHARED((NPAD,), f32),
            pltpu.SemaphoreType.DMA,
            pltpu.SemaphoreType.DMA,
            pltpu.SemaphoreType.DMA,
            pltpu.SemaphoreType.DMA,
            pltpu.SemaphoreType.DMA,
        ],
    )
    def k(table, elers, bvec, sd_in, acc_o, den_o,
          el_v, er_v, sd_v, ee_v, rows_v, bv_v, zv, acc_s, den_s,
          isem, gsem0, gsem1, ssem0, ssem1):
        cid = lax.axis_index("c")
        sid = lax.axis_index("s")
        gsem = (gsem0, gsem1)
        ssem = (ssem0, ssem1)

        def zero_b(i, c):
            zv[pl.ds(i * 16, 16)] = jnp.zeros((16,), f32)
            return c
        lax.fori_loop(0, RT // 16, zero_b, 0)

        def zero_rows(i, c):
            for cc in range(CN):
                rows_v[0, i, pl.ds(cc * 16, 16)] = jnp.zeros((16,), f32)
            return c
        lax.fori_loop(0, GG, zero_rows, 0)

        for j in range(RT // GG):
            pltpu.sync_copy(rows_v.at[0], acc_s.at[pl.ds(sid * RT + j * GG, GG)])
        pltpu.sync_copy(zv, den_s.at[pl.ds(sid * RT, RT)])

        pltpu.sync_copy(elers.at[cid].at[0], el_v)
        pltpu.sync_copy(elers.at[cid].at[1], er_v)
        pltpu.sync_copy(bvec.at[cid], bv_v)
        plsc.subcore_barrier()

        bv = bv_v[...]
        tbl = table.at[cid]
        base = sid * NG

        def ee_compute(p, b):
            def ee_b(kk, c):
                sv = sd_v[p, 0, pl.ds(kk * 16, 16)]
                dv = sd_v[p, 1, pl.ds(kk * 16, 16)]
                elg = plsc.load_gather(el_v, [sv])
                erg = plsc.load_gather(er_v, [dv])
                z = elg + erg
                z = jnp.maximum(z, z * 0.2)
                ee_v[b, pl.ds(kk * 16, 16)] = jnp.exp(z - bv)
                return c
            lax.fori_loop(0, KN, ee_b, 0)

        zeros16 = jnp.zeros((16,), i32)

        def scale(b):
            def scale_b(kk, c):
                kbase = kk * 16
                for j in range(16):
                    s = plsc.load_gather(ee_v.at[b], [zeros16 + (kbase + j)])
                    for cc in range(CN):
                        rows_v[b, kbase + j, pl.ds(cc * 16, 16)] = (
                            rows_v[b, kbase + j, pl.ds(cc * 16, 16)] * s)
                return c
            lax.fori_loop(0, KN, scale_b, 0)

        def scatter_descs(p, b):
            rowd = pltpu.make_async_copy(rows_v.at[b],
                                         acc_s.at[sd_v.at[p].at[1]], ssem[b])
            dend = pltpu.make_async_copy(ee_v.at[b],
                                         den_s.at[sd_v.at[p].at[1]], ssem[b])
            return rowd, dend

        def process(g, gp1_valid, drain_prev, b):
            nb = 1 - b
            for p in (b, b + 2):
                pq = (p + 1) % 4

                @pl.when(g % 4 == p)
                def _():
                    @pl.when(gp1_valid)
                    def _():
                        pltpu.async_copy(sd_in.at[base + g + 1], sd_v.at[pq],
                                         isem)
                    ee_compute(p, b)
                    pltpu.make_async_copy(tbl.at[sd_v.at[p].at[0]],
                                          rows_v.at[b], gsem[b]).wait()
                    scale(b)

                    @pl.when(drain_prev)
                    def _():
                        rd, dd = scatter_descs((p + 3) % 4, nb)
                        rd.wait()
                        dd.wait()

                    @pl.when(gp1_valid)
                    def _():
                        pltpu.make_async_copy(sd_in.at[base + g + 1],
                                              sd_v.at[pq], isem).wait()
                        pltpu.async_copy(tbl.at[sd_v.at[pq].at[0]],
                                         rows_v.at[nb], gsem[nb])

                    pltpu.async_copy(rows_v.at[b], acc_s.at[sd_v.at[p].at[1]],
                                     ssem[b], add=True)
                    pltpu.async_copy(ee_v.at[b], den_s.at[sd_v.at[p].at[1]],
                                     ssem[b], add=True)

        pltpu.sync_copy(sd_in.at[base], sd_v.at[0])
        pltpu.async_copy(tbl.at[sd_v.at[0].at[0]], rows_v.at[0], gsem[0])

        def pair(i, carry):
            g0 = i * 2
            process(g0, jnp.bool_(True), i > 0, 0)
            process(g0 + 1, i < NG // 2 - 1, jnp.bool_(True), 1)
            return carry
        lax.fori_loop(0, NG // 2, pair, 0)

        rd, dd = scatter_descs((NG - 1) % 4, 1)
        rd.wait()
        dd.wait()

        plsc.subcore_barrier()
        pltpu.sync_copy(acc_s.at[pl.ds(sid * RT, RT)],
                        acc_o.at[cid].at[pl.ds(sid * RT, RT)])
        pltpu.sync_copy(den_s.at[pl.ds(sid * RT, RT)],
                        den_o.at[cid].at[pl.ds(sid * RT, RT)])

    return k


_edge_sc_64 = _make_edge_sc(64)
_edge_sc_128 = _make_edge_sc(128)


def _prep_edges(src, dst, gg):
    pad = EPAD - E
    src_p = jnp.concatenate([src, (jnp.arange(pad, dtype=jnp.int32) * 97) % N])
    dst_p = jnp.concatenate([dst, N + jnp.arange(pad, dtype=jnp.int32) % _PADROWS])
    return jnp.stack([src_p.reshape(EPAD // gg, gg),
                      dst_p.reshape(EPAD // gg, gg)], axis=1)


def _bound_vec(m):
    b = m[0, 0] + m[0, 1]
    b = jnp.maximum(b, 0.2 * b)
    return jnp.full((16,), b, jnp.float32)


def kernel(x, edge_index0, edge_index1, W1, al1, ar1, b1, W2, al2, ar2, b2,
           W3, al3, ar3, b3, W4, al4, ar4, b4):
    sd0 = _prep_edges(edge_index0[0].astype(jnp.int32),
                      edge_index0[1].astype(jnp.int32), 128)
    sd1 = _prep_edges(edge_index1[0].astype(jnp.int32),
                      edge_index1[1].astype(jnp.int32), 64)
    xp = jnp.pad(x, ((0, NPAD - N), (0, 0)))

    t0, m1 = _dense_stage(xp, W1, al1, ar1, 64)
    t1, m3 = _dense_stage(xp, W3, al3, ar3, 64)
    elers_a = jnp.stack([jnp.stack([t0[:, 60], t0[:, 61]]),
                         jnp.stack([t1[:, 60], t1[:, 61]])])
    bvec_a = jnp.stack([_bound_vec(m1), _bound_vec(m3)])
    acc_a, den_a = _edge_sc_64(jnp.stack([t0, t1]), elers_a, bvec_a, sd0)

    out1 = _finish_stage(acc_a[0], den_a[0], b1, 60)
    out3 = _finish_stage(acc_a[1], den_a[1], b3, 60)

    t2, m2 = _dense_stage(out1, W2, al2, ar2, 128)
    t4, m4 = _dense_stage(out3, W4, al4, ar4, 128)
    elers_b = jnp.stack([jnp.stack([t2[:, 120], t2[:, 121]]),
                         jnp.stack([t4[:, 120], t4[:, 121]])])
    bvec_b = jnp.stack([_bound_vec(m2), _bound_vec(m4)])
    acc_b, den_b = _edge_sc_128(jnp.stack([t2, t4]), elers_b, bvec_b, sd1)

    h_a = _finish_stage(acc_b[0], den_b[0], b2, 120)
    h_f = _finish_stage(acc_b[1], den_b[1], b4, 120)
    return (h_a[:N], h_f[:N])


# speedup vs baseline: 82.0740x; 1.3167x over previous
import functools

import jax
import jax.numpy as jnp
from jax import lax
from jax.experimental import pallas as pl
from jax.experimental.pallas import tpu as pltpu
from jax.experimental.pallas import tpu_sc as plsc

N = 10000
E = 640000
NPAD = 10240
EPAD = 655360
ROW_BLK = 1024
_PADROWS = 240


def _dense_body(x_ref, w_ref, al_ref, ar_ref, o_ref, m_ref, *, cpad):
    i = pl.program_id(0)
    h = jnp.dot(x_ref[...], w_ref[...], preferred_element_type=jnp.float32)
    el = jnp.sum(h * al_ref[...], axis=1, keepdims=True)
    er = jnp.sum(h * ar_ref[...], axis=1, keepdims=True)
    pad = jnp.zeros((h.shape[0], cpad - h.shape[1] - 2), jnp.float32)
    o_ref[...] = jnp.concatenate([h, el, er, pad], axis=-1)
    cols = lax.broadcasted_iota(jnp.int32, (1, 128), 1)
    cur = jnp.where(cols == 0, jnp.max(el), jnp.where(cols == 1, jnp.max(er), -jnp.inf))

    @pl.when(i == 0)
    def _():
        m_ref[...] = cur

    @pl.when(i > 0)
    def _():
        m_ref[...] = jnp.maximum(m_ref[...], cur)


def _dense_stage(x, W, al, ar, cpad):
    din, dout = W.shape
    return pl.pallas_call(
        functools.partial(_dense_body, cpad=cpad),
        grid=(NPAD // ROW_BLK,),
        in_specs=[
            pl.BlockSpec((ROW_BLK, din), lambda i: (i, 0)),
            pl.BlockSpec((din, dout), lambda i: (0, 0)),
            pl.BlockSpec((1, dout), lambda i: (0, 0)),
            pl.BlockSpec((1, dout), lambda i: (0, 0)),
        ],
        out_specs=[
            pl.BlockSpec((ROW_BLK, cpad), lambda i: (i, 0)),
            pl.BlockSpec((1, 128), lambda i: (0, 0)),
        ],
        out_shape=[
            jax.ShapeDtypeStruct((NPAD, cpad), jnp.float32),
            jax.ShapeDtypeStruct((1, 128), jnp.float32),
        ],
    )(x, W, al.reshape(1, dout), ar.reshape(1, dout))


def _finish_body(acc_ref, den_ref, b_ref, o_ref):
    o_ref[...] = acc_ref[...] / (den_ref[...] + 1e-9) + b_ref[...]


def _finish_stage(acc, denom, b, dout):
    return pl.pallas_call(
        _finish_body,
        grid=(NPAD // ROW_BLK,),
        in_specs=[
            pl.BlockSpec((ROW_BLK, dout), lambda i: (i, 0)),
            pl.BlockSpec((ROW_BLK, 1), lambda i: (i, 0)),
            pl.BlockSpec((1, dout), lambda i: (0, 0)),
        ],
        out_specs=pl.BlockSpec((ROW_BLK, dout), lambda i: (i, 0)),
        out_shape=jax.ShapeDtypeStruct((NPAD, dout), jnp.float32),
    )(acc[:, :dout], denom.reshape(NPAD, 1), b.reshape(1, dout))


def _make_edge_sc(C):
    GG = 128 if C <= 64 else 64
    NG = (EPAD // GG) // 16
    KN = GG // 16
    CN = C // 16
    RT = NPAD // 16
    f32, i32 = jnp.float32, jnp.int32
    mesh = plsc.VectorSubcoreMesh(core_axis_name="c", subcore_axis_name="s")

    @functools.partial(
        pl.kernel,
        out_type=(
            jax.ShapeDtypeStruct((2, NPAD, C), f32),
            jax.ShapeDtypeStruct((2, NPAD), f32),
        ),
        mesh=mesh,
        compiler_params=pltpu.CompilerParams(needs_layout_passes=False,
                                             use_tc_tiling_on_sc=False),
        scratch_types=[
            pltpu.VMEM((NPAD,), f32),
            pltpu.VMEM((NPAD,), f32),
            pltpu.VMEM((4, 2, GG), i32),
            pltpu.VMEM((2, GG), f32),
            pltpu.VMEM((2, GG, C), f32),
            pltpu.VMEM((16,), f32),
            pltpu.VMEM((RT,), f32),
            pltpu.VMEM_SHARED((NPAD, C), f32),
            pltpu.VMEM_SHARED((NPAD,), f32),
            pltpu.SemaphoreType.DMA,
            pltpu.SemaphoreType.DMA,
            pltpu.SemaphoreType.DMA,
            pltpu.SemaphoreType.DMA,
            pltpu.SemaphoreType.DMA,
        ],
    )
    def k(table, elers, bvec, sd_in, acc_o, den_o,
          el_v, er_v, sd_v, ee_v, rows_v, bv_v, zv, acc_s, den_s,
          isem, gsem0, gsem1, ssem0, ssem1):
        cid = lax.axis_index("c")
        sid = lax.axis_index("s")
        gsem = (gsem0, gsem1)
        ssem = (ssem0, ssem1)

        def zero_b(i, c):
            zv[pl.ds(i * 16, 16)] = jnp.zeros((16,), f32)
            return c
        lax.fori_loop(0, RT // 16, zero_b, 0)

        def zero_rows(i, c):
            for cc in range(CN):
                rows_v[0, i, pl.ds(cc * 16, 16)] = jnp.zeros((16,), f32)
            return c
        lax.fori_loop(0, GG, zero_rows, 0)

        for j in range(RT // GG):
            pltpu.sync_copy(rows_v.at[0], acc_s.at[pl.ds(sid * RT + j * GG, GG)])
        pltpu.sync_copy(zv, den_s.at[pl.ds(sid * RT, RT)])

        pltpu.sync_copy(elers.at[cid].at[0], el_v)
        pltpu.sync_copy(elers.at[cid].at[1], er_v)
        pltpu.sync_copy(bvec.at[cid], bv_v)
        plsc.subcore_barrier()

        bv = bv_v[...]
        tbl = table.at[cid]
        base = sid * NG

        def ee_compute(p, b):
            @plsc.parallel_loop(0, KN, 1, unroll=2)
            def ee_b(kk):
                sv = sd_v[p, 0, pl.ds(kk * 16, 16)]
                dv = sd_v[p, 1, pl.ds(kk * 16, 16)]
                elg = plsc.load_gather(el_v, [sv])
                erg = plsc.load_gather(er_v, [dv])
                z = elg + erg
                z = jnp.maximum(z, z * 0.2)
                ee_v[b, pl.ds(kk * 16, 16)] = jnp.exp(z - bv)

        zeros16 = jnp.zeros((16,), i32)

        def scale(b):
            @plsc.parallel_loop(0, GG, 1, unroll=4)
            def scale_b(ei):
                s = plsc.load_gather(ee_v.at[b], [zeros16 + ei])
                for cc in range(CN):
                    rows_v[b, ei, pl.ds(cc * 16, 16)] = (
                        rows_v[b, ei, pl.ds(cc * 16, 16)] * s)

        def scatter_descs(p, b):
            rowd = pltpu.make_async_copy(rows_v.at[b],
                                         acc_s.at[sd_v.at[p].at[1]], ssem[b])
            dend = pltpu.make_async_copy(ee_v.at[b],
                                         den_s.at[sd_v.at[p].at[1]], ssem[b])
            return rowd, dend

        def process(g, gp1_valid, drain_prev, b):
            nb = 1 - b
            for p in (b, b + 2):
                pq = (p + 1) % 4

                @pl.when(g % 4 == p)
                def _():
                    @pl.when(gp1_valid)
                    def _():
                        pltpu.async_copy(sd_in.at[base + g + 1], sd_v.at[pq],
                                         isem)
                    ee_compute(p, b)
                    pltpu.make_async_copy(tbl.at[sd_v.at[p].at[0]],
                                          rows_v.at[b], gsem[b]).wait()
                    scale(b)

                    @pl.when(drain_prev)
                    def _():
                        rd, dd = scatter_descs((p + 3) % 4, nb)
                        rd.wait()
                        dd.wait()

                    @pl.when(gp1_valid)
                    def _():
                        pltpu.make_async_copy(sd_in.at[base + g + 1],
                                              sd_v.at[pq], isem).wait()
                        pltpu.async_copy(tbl.at[sd_v.at[pq].at[0]],
                                         rows_v.at[nb], gsem[nb])

                    pltpu.async_copy(rows_v.at[b], acc_s.at[sd_v.at[p].at[1]],
                                     ssem[b], add=True)
                    pltpu.async_copy(ee_v.at[b], den_s.at[sd_v.at[p].at[1]],
                                     ssem[b], add=True)

        pltpu.sync_copy(sd_in.at[base], sd_v.at[0])
        pltpu.async_copy(tbl.at[sd_v.at[0].at[0]], rows_v.at[0], gsem[0])

        def pair(i, carry):
            g0 = i * 2
            process(g0, jnp.bool_(True), i > 0, 0)
            process(g0 + 1, i < NG // 2 - 1, jnp.bool_(True), 1)
            return carry
        lax.fori_loop(0, NG // 2, pair, 0)

        rd, dd = scatter_descs((NG - 1) % 4, 1)
        rd.wait()
        dd.wait()

        plsc.subcore_barrier()
        pltpu.sync_copy(acc_s.at[pl.ds(sid * RT, RT)],
                        acc_o.at[cid].at[pl.ds(sid * RT, RT)])
        pltpu.sync_copy(den_s.at[pl.ds(sid * RT, RT)],
                        den_o.at[cid].at[pl.ds(sid * RT, RT)])

    return k


_edge_sc_64 = _make_edge_sc(64)
_edge_sc_128 = _make_edge_sc(128)


def _prep_edges(src, dst, gg):
    pad = EPAD - E
    src_p = jnp.concatenate([src, (jnp.arange(pad, dtype=jnp.int32) * 97) % N])
    dst_p = jnp.concatenate([dst, N + jnp.arange(pad, dtype=jnp.int32) % _PADROWS])
    return jnp.stack([src_p.reshape(EPAD // gg, gg),
                      dst_p.reshape(EPAD // gg, gg)], axis=1)


def _bound_vec(m):
    b = m[0, 0] + m[0, 1]
    b = jnp.maximum(b, 0.2 * b)
    return jnp.full((16,), b, jnp.float32)


def kernel(x, edge_index0, edge_index1, W1, al1, ar1, b1, W2, al2, ar2, b2,
           W3, al3, ar3, b3, W4, al4, ar4, b4):
    sd0 = _prep_edges(edge_index0[0].astype(jnp.int32),
                      edge_index0[1].astype(jnp.int32), 128)
    sd1 = _prep_edges(edge_index1[0].astype(jnp.int32),
                      edge_index1[1].astype(jnp.int32), 64)
    xp = jnp.pad(x, ((0, NPAD - N), (0, 0)))

    t0, m1 = _dense_stage(xp, W1, al1, ar1, 64)
    t1, m3 = _dense_stage(xp, W3, al3, ar3, 64)
    elers_a = jnp.stack([jnp.stack([t0[:, 60], t0[:, 61]]),
                         jnp.stack([t1[:, 60], t1[:, 61]])])
    bvec_a = jnp.stack([_bound_vec(m1), _bound_vec(m3)])
    acc_a, den_a = _edge_sc_64(jnp.stack([t0, t1]), elers_a, bvec_a, sd0)

    out1 = _finish_stage(acc_a[0], den_a[0], b1, 60)
    out3 = _finish_stage(acc_a[1], den_a[1], b3, 60)

    t2, m2 = _dense_stage(out1, W2, al2, ar2, 128)
    t4, m4 = _dense_stage(out3, W4, al4, ar4, 128)
    elers_b = jnp.stack([jnp.stack([t2[:, 120], t2[:, 121]]),
                         jnp.stack([t4[:, 120], t4[:, 121]])])
    bvec_b = jnp.stack([_bound_vec(m2), _bound_vec(m4)])
    acc_b, den_b = _edge_sc_128(jnp.stack([t2, t4]), elers_b, bvec_b, sd1)

    h_a = _finish_stage(acc_b[0], den_b[0], b2, 120)
    h_f = _finish_stage(acc_b[1], den_b[1], b4, 120)
    return (h_a[:N], h_f[:N])


# speedup vs baseline: 82.0774x; 1.0000x over previous
import functools

import jax
import jax.numpy as jnp
from jax import lax
from jax.experimental import pallas as pl
from jax.experimental.pallas import tpu as pltpu
from jax.experimental.pallas import tpu_sc as plsc

N = 10000
E = 640000
NPAD = 10240
EPAD = 655360
ROW_BLK = 1024
_PADROWS = 240


def _dense_body(x_ref, w_ref, al_ref, ar_ref, o_ref, m_ref, *, cpad):
    i = pl.program_id(0)
    h = jnp.dot(x_ref[...], w_ref[...], preferred_element_type=jnp.float32)
    el = jnp.sum(h * al_ref[...], axis=1, keepdims=True)
    er = jnp.sum(h * ar_ref[...], axis=1, keepdims=True)
    pad = jnp.zeros((h.shape[0], cpad - h.shape[1] - 2), jnp.float32)
    o_ref[...] = jnp.concatenate([h, el, er, pad], axis=-1)
    cols = lax.broadcasted_iota(jnp.int32, (1, 128), 1)
    cur = jnp.where(cols == 0, jnp.max(el), jnp.where(cols == 1, jnp.max(er), -jnp.inf))

    @pl.when(i == 0)
    def _():
        m_ref[...] = cur

    @pl.when(i > 0)
    def _():
        m_ref[...] = jnp.maximum(m_ref[...], cur)


def _dense_stage(x, W, al, ar, cpad):
    din, dout = W.shape
    return pl.pallas_call(
        functools.partial(_dense_body, cpad=cpad),
        grid=(NPAD // ROW_BLK,),
        in_specs=[
            pl.BlockSpec((ROW_BLK, din), lambda i: (i, 0)),
            pl.BlockSpec((din, dout), lambda i: (0, 0)),
            pl.BlockSpec((1, dout), lambda i: (0, 0)),
            pl.BlockSpec((1, dout), lambda i: (0, 0)),
        ],
        out_specs=[
            pl.BlockSpec((ROW_BLK, cpad), lambda i: (i, 0)),
            pl.BlockSpec((1, 128), lambda i: (0, 0)),
        ],
        out_shape=[
            jax.ShapeDtypeStruct((NPAD, cpad), jnp.float32),
            jax.ShapeDtypeStruct((1, 128), jnp.float32),
        ],
    )(x, W, al.reshape(1, dout), ar.reshape(1, dout))


def _finish_body(acc_ref, den_ref, b_ref, o_ref):
    o_ref[...] = acc_ref[...] / (den_ref[...] + 1e-9) + b_ref[...]


def _finish_stage(acc, denom, b, dout):
    return pl.pallas_call(
        _finish_body,
        grid=(NPAD // ROW_BLK,),
        in_specs=[
            pl.BlockSpec((ROW_BLK, dout), lambda i: (i, 0)),
            pl.BlockSpec((ROW_BLK, 1), lambda i: (i, 0)),
            pl.BlockSpec((1, dout), lambda i: (0, 0)),
        ],
        out_specs=pl.BlockSpec((ROW_BLK, dout), lambda i: (i, 0)),
        out_shape=jax.ShapeDtypeStruct((NPAD, dout), jnp.float32),
    )(acc[:, :dout], denom.reshape(NPAD, 1), b.reshape(1, dout))


def _make_edge_sc(C):
    GG = 128 if C <= 64 else 64
    NG = (EPAD // GG) // 16
    KN = GG // 16
    CN = C // 16
    RT = NPAD // 16
    f32, i32 = jnp.float32, jnp.int32
    mesh = plsc.VectorSubcoreMesh(core_axis_name="c", subcore_axis_name="s")

    @functools.partial(
        pl.kernel,
        out_type=(
            jax.ShapeDtypeStruct((2, NPAD, C), f32),
            jax.ShapeDtypeStruct((2, NPAD), f32),
        ),
        mesh=mesh,
        compiler_params=pltpu.CompilerParams(needs_layout_passes=False,
                                             use_tc_tiling_on_sc=False),
        scratch_types=[
            pltpu.VMEM((NPAD,), f32),
            pltpu.VMEM((NPAD,), f32),
            pltpu.VMEM((4, 2, GG), i32),
            pltpu.VMEM((2, GG), f32),
            pltpu.VMEM((2, GG, C), f32),
            pltpu.VMEM((16,), f32),
            pltpu.VMEM((RT,), f32),
            pltpu.VMEM_SHARED((NPAD, C), f32),
            pltpu.VMEM_SHARED((NPAD,), f32),
            pltpu.SemaphoreType.DMA,
            pltpu.SemaphoreType.DMA,
            pltpu.SemaphoreType.DMA,
            pltpu.SemaphoreType.DMA,
            pltpu.SemaphoreType.DMA,
        ],
    )
    def k(table, elers, bvec, sd_in, acc_o, den_o,
          el_v, er_v, sd_v, ee_v, rows_v, bv_v, zv, acc_s, den_s,
          isem, gsem0, gsem1, ssem0, ssem1):
        cid = lax.axis_index("c")
        sid = lax.axis_index("s")
        gsem = (gsem0, gsem1)
        ssem = (ssem0, ssem1)

        def zero_b(i, c):
            zv[pl.ds(i * 16, 16)] = jnp.zeros((16,), f32)
            return c
        lax.fori_loop(0, RT // 16, zero_b, 0)

        def zero_rows(i, c):
            for cc in range(CN):
                rows_v[0, i, pl.ds(cc * 16, 16)] = jnp.zeros((16,), f32)
            return c
        lax.fori_loop(0, GG, zero_rows, 0)

        for j in range(RT // GG):
            pltpu.sync_copy(rows_v.at[0], acc_s.at[pl.ds(sid * RT + j * GG, GG)])
        pltpu.sync_copy(zv, den_s.at[pl.ds(sid * RT, RT)])

        pltpu.sync_copy(elers.at[cid].at[0], el_v)
        pltpu.sync_copy(elers.at[cid].at[1], er_v)
        pltpu.sync_copy(bvec.at[cid], bv_v)
        plsc.subcore_barrier()

        bv = bv_v[...]
        tbl = table.at[cid]
        base = sid * NG

        def ee_compute(p, b):
            @plsc.parallel_loop(0, KN, 1, unroll=4)
            def ee_b(kk):
                sv = sd_v[p, 0, pl.ds(kk * 16, 16)]
                dv = sd_v[p, 1, pl.ds(kk * 16, 16)]
                elg = plsc.load_gather(el_v, [sv])
                erg = plsc.load_gather(er_v, [dv])
                z = elg + erg
                z = jnp.maximum(z, z * 0.2)
                ee_v[b, pl.ds(kk * 16, 16)] = jnp.exp(z - bv)

        zeros16 = jnp.zeros((16,), i32)

        def scale(b):
            @plsc.parallel_loop(0, GG, 1, unroll=8)
            def scale_b(ei):
                s = plsc.load_gather(ee_v.at[b], [zeros16 + ei])
                for cc in range(CN):
                    rows_v[b, ei, pl.ds(cc * 16, 16)] = (
                        rows_v[b, ei, pl.ds(cc * 16, 16)] * s)

        def scatter_descs(p, b):
            rowd = pltpu.make_async_copy(rows_v.at[b],
                                         acc_s.at[sd_v.at[p].at[1]], ssem[b])
            dend = pltpu.make_async_copy(ee_v.at[b],
                                         den_s.at[sd_v.at[p].at[1]], ssem[b])
            return rowd, dend

        def process(g, gp1_valid, drain_prev, b):
            nb = 1 - b
            for p in (b, b + 2):
                pq = (p + 1) % 4

                @pl.when(g % 4 == p)
                def _():
                    @pl.when(gp1_valid)
                    def _():
                        pltpu.async_copy(sd_in.at[base + g + 1], sd_v.at[pq],
                                         isem)
                    ee_compute(p, b)
                    pltpu.make_async_copy(tbl.at[sd_v.at[p].at[0]],
                                          rows_v.at[b], gsem[b]).wait()
                    scale(b)

                    @pl.when(drain_prev)
                    def _():
                        rd, dd = scatter_descs((p + 3) % 4, nb)
                        rd.wait()
                        dd.wait()

                    @pl.when(gp1_valid)
                    def _():
                        pltpu.make_async_copy(sd_in.at[base + g + 1],
                                              sd_v.at[pq], isem).wait()
                        pltpu.async_copy(tbl.at[sd_v.at[pq].at[0]],
                                         rows_v.at[nb], gsem[nb])

                    pltpu.async_copy(rows_v.at[b], acc_s.at[sd_v.at[p].at[1]],
                                     ssem[b], add=True)
                    pltpu.async_copy(ee_v.at[b], den_s.at[sd_v.at[p].at[1]],
                                     ssem[b], add=True)

        pltpu.sync_copy(sd_in.at[base], sd_v.at[0])
        pltpu.async_copy(tbl.at[sd_v.at[0].at[0]], rows_v.at[0], gsem[0])

        def pair(i, carry):
            g0 = i * 2
            process(g0, jnp.bool_(True), i > 0, 0)
            process(g0 + 1, i < NG // 2 - 1, jnp.bool_(True), 1)
            return carry
        lax.fori_loop(0, NG // 2, pair, 0)

        rd, dd = scatter_descs((NG - 1) % 4, 1)
        rd.wait()
        dd.wait()

        plsc.subcore_barrier()
        pltpu.sync_copy(acc_s.at[pl.ds(sid * RT, RT)],
                        acc_o.at[cid].at[pl.ds(sid * RT, RT)])
        pltpu.sync_copy(den_s.at[pl.ds(sid * RT, RT)],
                        den_o.at[cid].at[pl.ds(sid * RT, RT)])

    return k


_edge_sc_64 = _make_edge_sc(64)
_edge_sc_128 = _make_edge_sc(128)


def _prep_edges(src, dst, gg):
    pad = EPAD - E
    src_p = jnp.concatenate([src, (jnp.arange(pad, dtype=jnp.int32) * 97) % N])
    dst_p = jnp.concatenate([dst, N + jnp.arange(pad, dtype=jnp.int32) % _PADROWS])
    return jnp.stack([src_p.reshape(EPAD // gg, gg),
                      dst_p.reshape(EPAD // gg, gg)], axis=1)


def _bound_vec(m):
    b = m[0, 0] + m[0, 1]
    b = jnp.maximum(b, 0.2 * b)
    return jnp.full((16,), b, jnp.float32)


def kernel(x, edge_index0, edge_index1, W1, al1, ar1, b1, W2, al2, ar2, b2,
           W3, al3, ar3, b3, W4, al4, ar4, b4):
    sd0 = _prep_edges(edge_index0[0].astype(jnp.int32),
                      edge_index0[1].astype(jnp.int32), 128)
    sd1 = _prep_edges(edge_index1[0].astype(jnp.int32),
                      edge_index1[1].astype(jnp.int32), 64)
    xp = jnp.pad(x, ((0, NPAD - N), (0, 0)))

    t0, m1 = _dense_stage(xp, W1, al1, ar1, 64)
    t1, m3 = _dense_stage(xp, W3, al3, ar3, 64)
    elers_a = jnp.stack([jnp.stack([t0[:, 60], t0[:, 61]]),
                         jnp.stack([t1[:, 60], t1[:, 61]])])
    bvec_a = jnp.stack([_bound_vec(m1), _bound_vec(m3)])
    acc_a, den_a = _edge_sc_64(jnp.stack([t0, t1]), elers_a, bvec_a, sd0)

    out1 = _finish_stage(acc_a[0], den_a[0], b1, 60)
    out3 = _finish_stage(acc_a[1], den_a[1], b3, 60)

    t2, m2 = _dense_stage(out1, W2, al2, ar2, 128)
    t4, m4 = _dense_stage(out3, W4, al4, ar4, 128)
    elers_b = jnp.stack([jnp.stack([t2[:, 120], t2[:, 121]]),
                         jnp.stack([t4[:, 120], t4[:, 121]])])
    bvec_b = jnp.stack([_bound_vec(m2), _bound_vec(m4)])
    acc_b, den_b = _edge_sc_128(jnp.stack([t2, t4]), elers_b, bvec_b, sd1)

    h_a = _finish_stage(acc_b[0], den_b[0], b2, 120)
    h_f = _finish_stage(acc_b[1], den_b[1], b4, 120)
    return (h_a[:N], h_f[:N])


# speedup vs baseline: 83.9543x; 1.0229x over previous
import functools

import jax
import jax.numpy as jnp
from jax import lax
from jax.experimental import pallas as pl
from jax.experimental.pallas import tpu as pltpu
from jax.experimental.pallas import tpu_sc as plsc

N = 10000
E = 640000
NPAD = 10240
EPAD = 655360
ROW_BLK = 1024
_PADROWS = 240


def _emit_table(h, al_ref, ar_ref, o_ref, m_ref, i, cpad):
    el = jnp.sum(h * al_ref[0], axis=1, keepdims=True)
    er = jnp.sum(h * ar_ref[0], axis=1, keepdims=True)
    pad = jnp.zeros((h.shape[0], cpad - h.shape[1] - 2), jnp.float32)
    o_ref[0] = jnp.concatenate([h, el, er, pad], axis=-1)
    cols = lax.broadcasted_iota(jnp.int32, (1, 128), 1)
    cur = jnp.where(cols == 0, jnp.max(el), jnp.where(cols == 1, jnp.max(er), -jnp.inf))

    @pl.when(i == 0)
    def _():
        m_ref[0] = cur

    @pl.when(i > 0)
    def _():
        m_ref[0] = jnp.maximum(m_ref[0], cur)


def _stage1_body(x_ref, w_ref, al_ref, ar_ref, o_ref, m_ref, *, cpad):
    h = jnp.dot(x_ref[...], w_ref[0], preferred_element_type=jnp.float32)
    _emit_table(h, al_ref, ar_ref, o_ref, m_ref, pl.program_id(1), cpad)


def _stage1(x, Ws, als, ars, cpad):
    _, din, dout = Ws.shape
    return pl.pallas_call(
        functools.partial(_stage1_body, cpad=cpad),
        grid=(2, NPAD // ROW_BLK),
        in_specs=[
            pl.BlockSpec((ROW_BLK, din), lambda c, i: (i, 0)),
            pl.BlockSpec((1, din, dout), lambda c, i: (c, 0, 0)),
            pl.BlockSpec((1, 1, dout), lambda c, i: (c, 0, 0)),
            pl.BlockSpec((1, 1, dout), lambda c, i: (c, 0, 0)),
        ],
        out_specs=[
            pl.BlockSpec((1, ROW_BLK, cpad), lambda c, i: (c, i, 0)),
            pl.BlockSpec((1, 1, 128), lambda c, i: (c, 0, 0)),
        ],
        out_shape=[
            jax.ShapeDtypeStruct((2, NPAD, cpad), jnp.float32),
            jax.ShapeDtypeStruct((2, 1, 128), jnp.float32),
        ],
    )(x, Ws, als.reshape(2, 1, dout), ars.reshape(2, 1, dout))


def _stage2_body(acc_ref, b_ref, w_ref, al_ref, ar_ref, o_ref, m_ref,
                 *, dprev, cpad):
    a = acc_ref[0]
    out = a[:, :dprev] / (a[:, -1:] + 1e-9) + b_ref[0]
    h = jnp.dot(out, w_ref[0], preferred_element_type=jnp.float32)
    _emit_table(h, al_ref, ar_ref, o_ref, m_ref, pl.program_id(1), cpad)


def _stage2(acc, bs, Ws, als, ars, dprev, cpad):
    _, din, dout = Ws.shape
    cp = acc.shape[-1]
    return pl.pallas_call(
        functools.partial(_stage2_body, dprev=dprev, cpad=cpad),
        grid=(2, NPAD // ROW_BLK),
        in_specs=[
            pl.BlockSpec((1, ROW_BLK, cp), lambda c, i: (c, i, 0)),
            pl.BlockSpec((1, 1, dprev), lambda c, i: (c, 0, 0)),
            pl.BlockSpec((1, din, dout), lambda c, i: (c, 0, 0)),
            pl.BlockSpec((1, 1, dout), lambda c, i: (c, 0, 0)),
            pl.BlockSpec((1, 1, dout), lambda c, i: (c, 0, 0)),
        ],
        out_specs=[
            pl.BlockSpec((1, ROW_BLK, cpad), lambda c, i: (c, i, 0)),
            pl.BlockSpec((1, 1, 128), lambda c, i: (c, 0, 0)),
        ],
        out_shape=[
            jax.ShapeDtypeStruct((2, NPAD, cpad), jnp.float32),
            jax.ShapeDtypeStruct((2, 1, 128), jnp.float32),
        ],
    )(acc, bs.reshape(2, 1, dprev), Ws, als.reshape(2, 1, dout),
      ars.reshape(2, 1, dout))


def _final_body(acc_ref, b_ref, o_ref, *, dprev):
    a = acc_ref[0]
    o_ref[0] = a[:, :dprev] / (a[:, -1:] + 1e-9) + b_ref[0]


def _final(acc, bs, dprev):
    cp = acc.shape[-1]
    return pl.pallas_call(
        functools.partial(_final_body, dprev=dprev),
        grid=(2, NPAD // ROW_BLK),
        in_specs=[
            pl.BlockSpec((1, ROW_BLK, cp), lambda c, i: (c, i, 0)),
            pl.BlockSpec((1, 1, dprev), lambda c, i: (c, 0, 0)),
        ],
        out_specs=pl.BlockSpec((1, ROW_BLK, dprev), lambda c, i: (c, i, 0)),
        out_shape=jax.ShapeDtypeStruct((2, NPAD, dprev), jnp.float32),
    )(acc, bs.reshape(2, 1, dprev))


def _make_edge_sc(C):
    GG = 128 if C <= 64 else 64
    NG = (EPAD // GG) // 16
    KN = GG // 16
    CN = C // 16
    RT = NPAD // 16
    f32, i32 = jnp.float32, jnp.int32
    mesh = plsc.VectorSubcoreMesh(core_axis_name="c", subcore_axis_name="s")

    @functools.partial(
        pl.kernel,
        out_type=jax.ShapeDtypeStruct((2, NPAD, C), f32),
        mesh=mesh,
        compiler_params=pltpu.CompilerParams(needs_layout_passes=False,
                                             use_tc_tiling_on_sc=False),
        scratch_types=[
            pltpu.VMEM((NPAD,), f32),
            pltpu.VMEM((NPAD,), f32),
            pltpu.VMEM((4, 2, GG), i32),
            pltpu.VMEM((2, GG), f32),
            pltpu.VMEM((2, GG, C), f32),
            pltpu.VMEM((16,), f32),
            pltpu.VMEM_SHARED((NPAD, C), f32),
            pltpu.SemaphoreType.DMA,
            pltpu.SemaphoreType.DMA,
            pltpu.SemaphoreType.DMA,
            pltpu.SemaphoreType.DMA,
            pltpu.SemaphoreType.DMA,
        ],
    )
    def k(table, elers, bvec, sd_in, acc_o,
          el_v, er_v, sd_v, ee_v, rows_v, bv_v, acc_s,
          isem, gsem0, gsem1, ssem0, ssem1):
        cid = lax.axis_index("c")
        sid = lax.axis_index("s")
        gsem = (gsem0, gsem1)
        ssem = (ssem0, ssem1)

        def zero_rows(i, c):
            for cc in range(CN):
                rows_v[0, i, pl.ds(cc * 16, 16)] = jnp.zeros((16,), f32)
            return c
        lax.fori_loop(0, GG, zero_rows, 0)

        for j in range(RT // GG):
            pltpu.sync_copy(rows_v.at[0], acc_s.at[pl.ds(sid * RT + j * GG, GG)])

        pltpu.sync_copy(elers.at[cid].at[0], el_v)
        pltpu.sync_copy(elers.at[cid].at[1], er_v)
        pltpu.sync_copy(bvec.at[cid], bv_v)
        plsc.subcore_barrier()

        bv = bv_v[...]
        tbl = table.at[cid]
        base = sid * NG

        def ee_compute(p, b):
            @plsc.parallel_loop(0, KN, 1, unroll=4)
            def ee_b(kk):
                sv = sd_v[p, 0, pl.ds(kk * 16, 16)]
                dv = sd_v[p, 1, pl.ds(kk * 16, 16)]
                elg = plsc.load_gather(el_v, [sv])
                erg = plsc.load_gather(er_v, [dv])
                z = elg + erg
                z = jnp.maximum(z, z * 0.2)
                ee_v[b, pl.ds(kk * 16, 16)] = jnp.exp(z - bv)

        zeros16 = jnp.zeros((16,), i32)

        def scale(b):
            @plsc.parallel_loop(0, GG, 1, unroll=8)
            def scale_b(ei):
                s = plsc.load_gather(ee_v.at[b], [zeros16 + ei])
                for cc in range(CN):
                    rows_v[b, ei, pl.ds(cc * 16, 16)] = (
                        rows_v[b, ei, pl.ds(cc * 16, 16)] * s)

        iota16 = lax.broadcasted_iota(i32, (16,), 0)

        def write_ee_col(b):
            def wc(kk, c):
                eev = ee_v[b, pl.ds(kk * 16, 16)]
                plsc.store_scatter(
                    rows_v, [zeros16 + b, kk * 16 + iota16, zeros16 + (C - 1)],
                    eev)
                return c
            lax.fori_loop(0, KN, wc, 0)

        def scatter_desc(p, b):
            return pltpu.make_async_copy(rows_v.at[b],
                                         acc_s.at[sd_v.at[p].at[1]], ssem[b])

        def process(g, gp1_valid, drain_prev, b):
            nb = 1 - b
            for p in (b, b + 2):
                pq = (p + 1) % 4

                @pl.when(g % 4 == p)
                def _():
                    @pl.when(gp1_valid)
                    def _():
                        pltpu.async_copy(sd_in.at[base + g + 1], sd_v.at[pq],
                                         isem)
                    ee_compute(p, b)
                    pltpu.make_async_copy(tbl.at[sd_v.at[p].at[0]],
                                          rows_v.at[b], gsem[b]).wait()
                    scale(b)
                    write_ee_col(b)

                    @pl.when(drain_prev)
                    def _():
                        scatter_desc((p + 3) % 4, nb).wait()

                    @pl.when(gp1_valid)
                    def _():
                        pltpu.make_async_copy(sd_in.at[base + g + 1],
                                              sd_v.at[pq], isem).wait()
                        pltpu.async_copy(tbl.at[sd_v.at[pq].at[0]],
                                         rows_v.at[nb], gsem[nb])

                    pltpu.async_copy(rows_v.at[b], acc_s.at[sd_v.at[p].at[1]],
                                     ssem[b], add=True)

        pltpu.sync_copy(sd_in.at[base], sd_v.at[0])
        pltpu.async_copy(tbl.at[sd_v.at[0].at[0]], rows_v.at[0], gsem[0])

        def pair(i, carry):
            g0 = i * 2
            process(g0, jnp.bool_(True), i > 0, 0)
            process(g0 + 1, i < NG // 2 - 1, jnp.bool_(True), 1)
            return carry
        lax.fori_loop(0, NG // 2, pair, 0)

        scatter_desc((NG - 1) % 4, 1).wait()

        plsc.subcore_barrier()
        pltpu.sync_copy(acc_s.at[pl.ds(sid * RT, RT)],
                        acc_o.at[cid].at[pl.ds(sid * RT, RT)])

    return k


_edge_sc_64 = _make_edge_sc(64)
_edge_sc_128 = _make_edge_sc(128)


def _prep_edges(src, dst, gg):
    pad = EPAD - E
    src_p = jnp.concatenate([src, (jnp.arange(pad, dtype=jnp.int32) * 97) % N])
    dst_p = jnp.concatenate([dst, N + jnp.arange(pad, dtype=jnp.int32) % _PADROWS])
    return jnp.stack([src_p.reshape(EPAD // gg, gg),
                      dst_p.reshape(EPAD // gg, gg)], axis=1)


def _bound_vecs(m):
    b = m[:, 0, 0] + m[:, 0, 1]
    b = jnp.maximum(b, 0.2 * b)
    return jnp.broadcast_to(b[:, None], (2, 16))


def _elers(t, dcol):
    return jnp.swapaxes(t[:, :, dcol:dcol + 2], 1, 2)


def kernel(x, edge_index0, edge_index1, W1, al1, ar1, b1, W2, al2, ar2, b2,
           W3, al3, ar3, b3, W4, al4, ar4, b4):
    sd0 = _prep_edges(edge_index0[0].astype(jnp.int32),
                      edge_index0[1].astype(jnp.int32), 128)
    sd1 = _prep_edges(edge_index1[0].astype(jnp.int32),
                      edge_index1[1].astype(jnp.int32), 64)
    xp = jnp.pad(x, ((0, NPAD - N), (0, 0)))

    t_a, m_a = _stage1(xp, jnp.stack([W1, W3]), jnp.stack([al1, al3]),
                       jnp.stack([ar1, ar3]), 64)
    acc_a = _edge_sc_64(t_a, _elers(t_a, 60), _bound_vecs(m_a), sd0)

    t_b, m_b = _stage2(acc_a, jnp.stack([b1, b3]), jnp.stack([W2, W4]),
                       jnp.stack([al2, al4]), jnp.stack([ar2, ar4]), 60, 128)
    acc_b = _edge_sc_128(t_b, _elers(t_b, 120), _bound_vecs(m_b), sd1)

    out = _final(acc_b, jnp.stack([b2, b4]), 120)
    return (out[0, :N], out[1, :N])
